# baseline (speedup 1.0000x reference)
import functools

import jax
import jax.numpy as jnp
from jax import lax
from jax.experimental import pallas as pl
from jax.experimental.pallas import tpu as pltpu
from jax.experimental.pallas import tpu_sc as plsc

_NC = 2
_NS = 16
_NW = _NC * _NS
_L = 16

_EMBED = 64
_WB_LEN = 144


def _sc_body(users_r, movies_r, utab, mtab, wb, out,
             idx_u, idx_m, rows_u, rows_m, out_v, w_v, sem,
             *, b_per_w, n_blocks):
    wid = lax.axis_index("s") * _NC + lax.axis_index("c")

    pltpu.sync_copy(users_r.at[wid], idx_u)
    pltpu.sync_copy(movies_r.at[wid], idx_m)
    pltpu.sync_copy(wb, w_v)

    cps = []
    for j in range(n_blocks):
        dst = pl.ds(j * 128, 128)
        cps.append(pltpu.async_copy(utab.at[idx_u.at[j]], rows_u.at[dst], sem))
        cps.append(pltpu.async_copy(mtab.at[idx_m.at[j]], rows_m.at[dst], sem))
    for cp in cps:
        cp.wait()

    w_regs = [w_v[pl.ds(k * _L, _L)] for k in range(2 * _EMBED // _L)]
    bias = w_v[pl.ds(2 * _EMBED, _L)][0]
    row16 = lax.iota(jnp.int32, _L)
    n_chunks = b_per_w // _L

    def chunk(i, carry):
        rid = i * _L + row16
        accs = [jnp.zeros((_L,), jnp.float32) for _ in range(4)]
        for d in range(_EMBED):
            col = jnp.full((_L,), d, jnp.int32)
            v = plsc.load_gather(rows_u, [rid, col])
            accs[d % 4] = accs[d % 4] + v * w_regs[d // _L][d % _L]
        for d in range(_EMBED):
            col = jnp.full((_L,), d, jnp.int32)
            v = plsc.load_gather(rows_m, [rid, col])
            dd = _EMBED + d
            accs[d % 4] = accs[d % 4] + v * w_regs[dd // _L][dd % _L]
        acc = (accs[0] + accs[1]) + (accs[2] + accs[3]) + bias
        out_v[pl.ds(i * _L, _L)] = acc
        return carry

    lax.fori_loop(0, n_chunks, chunk, 0)
    pltpu.sync_copy(out_v, out.at[wid])


def kernel(users, movies, user_table, movie_table, W, b):
    B = users.shape[0]
    assert B % (_NW * 128) == 0
    b_per_w = B // _NW
    n_blocks = b_per_w // 128

    users_r = users.astype(jnp.int32).reshape(_NW, n_blocks, 128)
    movies_r = movies.astype(jnp.int32).reshape(_NW, n_blocks, 128)
    wb = jnp.zeros((_WB_LEN,), jnp.float32)
    wb = wb.at[: 2 * _EMBED].set(W.reshape(-1).astype(jnp.float32))
    wb = wb.at[2 * _EMBED].set(b.reshape(()).astype(jnp.float32))

    mesh = plsc.VectorSubcoreMesh(core_axis_name="c", subcore_axis_name="s")
    body = functools.partial(_sc_body, b_per_w=b_per_w, n_blocks=n_blocks)
    run = pl.kernel(
        body,
        out_type=jax.ShapeDtypeStruct((_NW, b_per_w), jnp.float32),
        mesh=mesh,
        compiler_params=pltpu.CompilerParams(
            needs_layout_passes=False,
            use_tc_tiling_on_sc=False,
        ),
        scratch_types=[
            pltpu.VMEM((n_blocks, 128), jnp.int32),
            pltpu.VMEM((n_blocks, 128), jnp.int32),
            pltpu.VMEM((b_per_w, _EMBED), jnp.float32),
            pltpu.VMEM((b_per_w, _EMBED), jnp.float32),
            pltpu.VMEM((b_per_w,), jnp.float32),
            pltpu.VMEM((_WB_LEN,), jnp.float32),
            pltpu.SemaphoreType.DMA,
        ],
    )
    out = run(users_r, movies_r, user_table, movie_table, wb)
    return out.reshape(B, 1)


# speedup vs baseline: 1.0169x; 1.0169x over previous
import functools

import jax
import jax.numpy as jnp
from jax import lax
from jax.experimental import pallas as pl
from jax.experimental.pallas import tpu as pltpu
from jax.experimental.pallas import tpu_sc as plsc

_NC = 2
_NS = 16
_NW = _NC * _NS
_L = 16

_EMBED = 64
_BLK = 128


def _sc_body(idxp_u, idxp_m, hu_r, hm_r, utab, mtab, wsk_u, wsk_m, out,
             idx_u, idx_m, hu_v, hm_v, rows_u, rows_m, out_v, wu_v, wm_v,
             sem_u0, sem_u1, sem_m0, sem_m1,
             *, b_per_w, n_blocks):
    wid = lax.axis_index("s") * _NC + lax.axis_index("c")

    pltpu.sync_copy(idxp_u.at[wid], idx_u)
    pltpu.sync_copy(idxp_m.at[wid], idx_m)
    pltpu.sync_copy(hu_r.at[wid], hu_v)
    pltpu.sync_copy(hm_r.at[wid], hm_v)
    pltpu.sync_copy(wsk_u, wu_v)
    pltpu.sync_copy(wsk_m, wm_v)

    sems_u = (sem_u0, sem_u1)
    sems_m = (sem_m0, sem_m1)

    def fire(j):
        s = j % 2
        return (
            pltpu.async_copy(utab.at[idx_u.at[j]], rows_u.at[s], sems_u[s]),
            pltpu.async_copy(mtab.at[idx_m.at[j]], rows_m.at[s], sems_m[s]),
        )

    row16 = lax.iota(jnp.int32, _L)
    bias = wu_v[pl.ds(_EMBED * _L, _L)]
    chunks_per_blk = _BLK // _L

    inflight = {0: fire(0), 1: fire(1)}
    for j in range(n_blocks):
        s = j % 2
        cu, cm = inflight.pop(j)
        cu.wait()
        cm.wait()
        ru = rows_u.at[s]
        rm = rows_m.at[s]

        def chunk(k, carry, *, j=j, ru=ru, rm=rm):
            g = j * chunks_per_blk + k
            hu_off = hu_v[pl.ds(g * _L, _L)] << 6
            hm_off = hm_v[pl.ds(g * _L, _L)] << 6
            rid = k * _L + row16
            accs = [jnp.zeros((_L,), jnp.float32) for _ in range(4)]
            for d in range(_EMBED):
                t = (row16 + d) & 63
                wu = wu_v[pl.ds(d * _L, _L)]
                wm = wm_v[pl.ds(d * _L, _L)]
                vu = plsc.load_gather(ru, [rid, hu_off + t])
                vm = plsc.load_gather(rm, [rid, hm_off + t])
                accs[d % 4] = accs[d % 4] + (vu * wu + vm * wm)
            acc = (accs[0] + accs[1]) + (accs[2] + accs[3]) + bias
            out_v[pl.ds(g * _L, _L)] = acc
            return carry

        lax.fori_loop(0, chunks_per_blk, chunk, 0)
        if j + 2 < n_blocks:
            inflight[j + 2] = fire(j + 2)

    pltpu.sync_copy(out_v, out.at[wid])


def kernel(users, movies, user_table, movie_table, W, b):
    B = users.shape[0]
    assert B % (_NW * _BLK) == 0
    b_per_w = B // _NW
    n_blocks = b_per_w // _BLK
    V = user_table.shape[0]

    users = users.astype(jnp.int32)
    movies = movies.astype(jnp.int32)
    idxp_u = (users >> 1).reshape(_NW, n_blocks, _BLK)
    idxp_m = (movies >> 1).reshape(_NW, n_blocks, _BLK)
    hu = (users & 1).reshape(_NW, b_per_w)
    hm = (movies & 1).reshape(_NW, b_per_w)

    utab = user_table.reshape(V // 2, 2 * _EMBED)
    mtab = movie_table.reshape(V // 2, 2 * _EMBED)

    rot = (jnp.arange(_EMBED)[:, None] + jnp.arange(_L)[None, :]) & 63
    wu = W.reshape(-1)[:_EMBED].astype(jnp.float32)
    wm = W.reshape(-1)[_EMBED:].astype(jnp.float32)
    wsk_u = jnp.concatenate(
        [wu[rot].reshape(-1), jnp.full((_L,), b.reshape(()), dtype=jnp.float32)]
    )
    wsk_m = wm[rot].reshape(-1)

    mesh = plsc.VectorSubcoreMesh(core_axis_name="c", subcore_axis_name="s")
    body = functools.partial(_sc_body, b_per_w=b_per_w, n_blocks=n_blocks)
    run = pl.kernel(
        body,
        out_type=jax.ShapeDtypeStruct((_NW, b_per_w), jnp.float32),
        mesh=mesh,
        compiler_params=pltpu.CompilerParams(needs_layout_passes=False),
        scratch_types=[
            pltpu.VMEM((n_blocks, _BLK), jnp.int32),
            pltpu.VMEM((n_blocks, _BLK), jnp.int32),
            pltpu.VMEM((b_per_w,), jnp.int32),
            pltpu.VMEM((b_per_w,), jnp.int32),
            pltpu.VMEM((2, _BLK, 2 * _EMBED), jnp.float32),
            pltpu.VMEM((2, _BLK, 2 * _EMBED), jnp.float32),
            pltpu.VMEM((b_per_w,), jnp.float32),
            pltpu.VMEM((_EMBED * _L + _L,), jnp.float32),
            pltpu.VMEM((_EMBED * _L,), jnp.float32),
            pltpu.SemaphoreType.DMA,
            pltpu.SemaphoreType.DMA,
            pltpu.SemaphoreType.DMA,
            pltpu.SemaphoreType.DMA,
        ],
    )
    out = run(idxp_u, idxp_m, hu, hm, utab, mtab, wsk_u, wsk_m)
    return out.reshape(B, 1)


# speedup vs baseline: 1.5001x; 1.4751x over previous
import functools

import jax
import jax.numpy as jnp
from jax import lax
from jax.experimental import pallas as pl
from jax.experimental.pallas import tpu as pltpu
from jax.experimental.pallas import tpu_sc as plsc

_NC = 2
_NS = 16
_NW = _NC * _NS
_L = 16

_EMBED = 64
_BLK = 128


def _sc_body(users_r, movies_r, utab, mtab, wsk_u, wsk_m, out,
             idx_u, idx_m, rows_u, rows_m, out_v, wu_v, wm_v,
             sem_u0, sem_u1, sem_m0, sem_m1,
             *, b_per_w, n_blocks):
    wid = lax.axis_index("s") * _NC + lax.axis_index("c")
    chunks_per_blk = _BLK // _L

    pltpu.sync_copy(users_r.at[wid], idx_u)
    pltpu.sync_copy(movies_r.at[wid], idx_m)
    pltpu.sync_copy(wsk_u, wu_v)
    pltpu.sync_copy(wsk_m, wm_v)

    sems_u = (sem_u0, sem_u1)
    sems_m = (sem_m0, sem_m1)

    def fire(j):
        s = j % 2
        ru = rows_u.at[s]
        rm = rows_m.at[s]

        def issue_chunk(c, carry):
            base = j * _BLK + c * _L
            iu = idx_u[pl.ds(base, _L)]
            im = idx_m[pl.ds(base, _L)]
            for l in range(_L):
                r = c * _L + l
                pltpu.async_copy(utab.at[pl.ds(iu[l], 1)],
                                 ru.at[pl.ds(r, 1)], sems_u[s])
                pltpu.async_copy(mtab.at[pl.ds(im[l], 1)],
                                 rm.at[pl.ds(r, 1)], sems_m[s])
            return carry

        lax.fori_loop(0, chunks_per_blk, issue_chunk, 0)

    def drain(j):
        s = j % 2
        pltpu.make_async_copy(utab.at[pl.ds(0, _BLK)], rows_u.at[s],
                              sems_u[s]).wait()
        pltpu.make_async_copy(mtab.at[pl.ds(0, _BLK)], rows_m.at[s],
                              sems_m[s]).wait()

    row16 = lax.iota(jnp.int32, _L)
    bias = wu_v[pl.ds(_EMBED * _L, _L)]

    fire(0)
    fire(1)
    for j in range(n_blocks):
        s = j % 2
        drain(j)
        ru = rows_u.at[s]
        rm = rows_m.at[s]

        def chunk(k, carry, *, j=j, ru=ru, rm=rm):
            g = j * chunks_per_blk + k
            rid = k * _L + row16
            accs = [jnp.zeros((_L,), jnp.float32) for _ in range(4)]
            for d in range(_EMBED):
                t = (row16 + d) & 63
                wu = wu_v[pl.ds(d * _L, _L)]
                wm = wm_v[pl.ds(d * _L, _L)]
                vu = plsc.load_gather(ru, [rid, t])
                vm = plsc.load_gather(rm, [rid, t])
                accs[d % 4] = accs[d % 4] + (vu * wu + vm * wm)
            acc = (accs[0] + accs[1]) + (accs[2] + accs[3]) + bias
            out_v[pl.ds(g * _L, _L)] = acc
            return carry

        lax.fori_loop(0, chunks_per_blk, chunk, 0)
        if j + 2 < n_blocks:
            fire(j + 2)

    pltpu.sync_copy(out_v, out.at[wid])


def kernel(users, movies, user_table, movie_table, W, b):
    B = users.shape[0]
    assert B % (_NW * _BLK) == 0
    b_per_w = B // _NW
    n_blocks = b_per_w // _BLK

    users_r = users.astype(jnp.int32).reshape(_NW, b_per_w)
    movies_r = movies.astype(jnp.int32).reshape(_NW, b_per_w)

    rot = (jnp.arange(_EMBED)[:, None] + jnp.arange(_L)[None, :]) & 63
    wu = W.reshape(-1)[:_EMBED].astype(jnp.float32)
    wm = W.reshape(-1)[_EMBED:].astype(jnp.float32)
    wsk_u = jnp.concatenate(
        [wu[rot].reshape(-1), jnp.full((_L,), b.reshape(()), dtype=jnp.float32)]
    )
    wsk_m = wm[rot].reshape(-1)

    mesh = plsc.VectorSubcoreMesh(core_axis_name="c", subcore_axis_name="s")
    body = functools.partial(_sc_body, b_per_w=b_per_w, n_blocks=n_blocks)
    run = pl.kernel(
        body,
        out_type=jax.ShapeDtypeStruct((_NW, b_per_w), jnp.float32),
        mesh=mesh,
        compiler_params=pltpu.CompilerParams(needs_layout_passes=False),
        scratch_types=[
            pltpu.VMEM((b_per_w,), jnp.int32),
            pltpu.VMEM((b_per_w,), jnp.int32),
            pltpu.VMEM((2, _BLK, _EMBED), jnp.float32),
            pltpu.VMEM((2, _BLK, _EMBED), jnp.float32),
            pltpu.VMEM((b_per_w,), jnp.float32),
            pltpu.VMEM((_EMBED * _L + _L,), jnp.float32),
            pltpu.VMEM((_EMBED * _L,), jnp.float32),
            pltpu.SemaphoreType.DMA,
            pltpu.SemaphoreType.DMA,
            pltpu.SemaphoreType.DMA,
            pltpu.SemaphoreType.DMA,
        ],
    )
    out = run(users_r, movies_r, user_table, movie_table, wsk_u, wsk_m)
    return out.reshape(B, 1)


# speedup vs baseline: 3.8937x; 2.5955x over previous
import functools

import jax
import jax.numpy as jnp
from jax import lax
from jax.experimental import pallas as pl
from jax.experimental.pallas import tpu as pltpu
from jax.experimental.pallas import tpu_sc as plsc

_NC = 2
_NS = 16
_NW = _NC * _NS
_L = 16

_EMBED = 64
_BLK = 64
_C = 8192


def _tc_scores_body(x_ref, w_ref, o_ref):
    x = x_ref[...]
    w = w_ref[...]
    s = jnp.sum(x * w, axis=0)
    o_ref[...] = s.reshape(_C // 128, 128)


def _tc_scores(tab_t, w):
    n = tab_t.shape[1]
    grid = (n + _C - 1) // _C
    return pl.pallas_call(
        _tc_scores_body,
        grid=(grid,),
        in_specs=[
            pl.BlockSpec((_EMBED, _C), lambda i: (0, i)),
            pl.BlockSpec((_EMBED, 1), lambda i: (0, 0)),
        ],
        out_specs=pl.BlockSpec((_C // 128, 128), lambda i: (i, 0)),
        out_shape=jax.ShapeDtypeStruct((grid * (_C // 128), 128), jnp.float32),
    )(tab_t, w)


def _sc_body(qu_r, tu_r, qm_r, tm_r, su, sm, brep, out,
             qu_v, tu_v, qm_v, tm_v, rows_u, rows_m, out_v, b_v,
             sem_u0, sem_u1, sem_m0, sem_m1,
             *, b_per_w, n_blocks):
    wid = lax.axis_index("s") * _NC + lax.axis_index("c")
    chunks_per_blk = _BLK // _L

    pltpu.sync_copy(qu_r.at[wid], qu_v)
    pltpu.sync_copy(tu_r.at[wid], tu_v)
    pltpu.sync_copy(qm_r.at[wid], qm_v)
    pltpu.sync_copy(tm_r.at[wid], tm_v)
    pltpu.sync_copy(brep, b_v)

    sems_u = (sem_u0, sem_u1)
    sems_m = (sem_m0, sem_m1)

    def fire(j):
        s = j % 2
        blk = pl.ds(j * _BLK, _BLK)
        pltpu.async_copy(su.at[qu_v.at[blk]], rows_u.at[s], sems_u[s])
        pltpu.async_copy(sm.at[qm_v.at[blk]], rows_m.at[s], sems_m[s])

    def drain(j):
        s = j % 2
        pltpu.make_async_copy(su.at[pl.ds(0, _BLK)], rows_u.at[s],
                              sems_u[s]).wait()
        pltpu.make_async_copy(sm.at[pl.ds(0, _BLK)], rows_m.at[s],
                              sems_m[s]).wait()

    row16 = lax.iota(jnp.int32, _L)
    bias = b_v[...]

    fire(0)
    fire(1)
    for j in range(n_blocks):
        s = j % 2
        drain(j)
        ru = rows_u.at[s]
        rm = rows_m.at[s]

        def chunk(k, carry, *, j=j, ru=ru, rm=rm):
            g = j * chunks_per_blk + k
            rid = k * _L + row16
            t_u = tu_v[pl.ds(g * _L, _L)]
            t_m = tm_v[pl.ds(g * _L, _L)]
            vu = plsc.load_gather(ru, [rid, t_u])
            vm = plsc.load_gather(rm, [rid, t_m])
            out_v[pl.ds(g * _L, _L)] = vu + vm + bias
            return carry

        lax.fori_loop(0, chunks_per_blk, chunk, 0)
        if j + 2 < n_blocks:
            fire(j + 2)

    pltpu.sync_copy(out_v, out.at[wid])


def kernel(users, movies, user_table, movie_table, W, b):
    B = users.shape[0]
    assert B % (_NW * _BLK) == 0
    b_per_w = B // _NW
    n_blocks = b_per_w // _BLK

    users = users.astype(jnp.int32)
    movies = movies.astype(jnp.int32)
    qu = (users >> 7).reshape(_NW, b_per_w)
    tu = (users & 127).reshape(_NW, b_per_w)
    qm = (movies >> 7).reshape(_NW, b_per_w)
    tm = (movies & 127).reshape(_NW, b_per_w)

    utab_t = user_table.T
    mtab_t = movie_table.T
    wu = W.reshape(-1)[:_EMBED].astype(jnp.float32).reshape(_EMBED, 1)
    wm = W.reshape(-1)[_EMBED:].astype(jnp.float32).reshape(_EMBED, 1)

    scores_u = _tc_scores(utab_t, wu)
    scores_m = _tc_scores(mtab_t, wm)

    brep = jnp.full((_L,), b.reshape(()), dtype=jnp.float32)

    mesh = plsc.VectorSubcoreMesh(core_axis_name="c", subcore_axis_name="s")
    body = functools.partial(_sc_body, b_per_w=b_per_w, n_blocks=n_blocks)
    run = pl.kernel(
        body,
        out_type=jax.ShapeDtypeStruct((_NW, b_per_w), jnp.float32),
        mesh=mesh,
        compiler_params=pltpu.CompilerParams(needs_layout_passes=False),
        scratch_types=[
            pltpu.VMEM((b_per_w,), jnp.int32),
            pltpu.VMEM((b_per_w,), jnp.int32),
            pltpu.VMEM((b_per_w,), jnp.int32),
            pltpu.VMEM((b_per_w,), jnp.int32),
            pltpu.VMEM((2, _BLK, 128), jnp.float32),
            pltpu.VMEM((2, _BLK, 128), jnp.float32),
            pltpu.VMEM((b_per_w,), jnp.float32),
            pltpu.VMEM((_L,), jnp.float32),
            pltpu.SemaphoreType.DMA,
            pltpu.SemaphoreType.DMA,
            pltpu.SemaphoreType.DMA,
            pltpu.SemaphoreType.DMA,
        ],
    )
    out = run(qu, tu, qm, tm, scores_u, scores_m, brep)
    return out.reshape(B, 1)


# speedup vs baseline: 5.8127x; 1.4929x over previous
import functools

import jax
import jax.numpy as jnp
from jax import lax
from jax.experimental import pallas as pl
from jax.experimental.pallas import tpu as pltpu
from jax.experimental.pallas import tpu_sc as plsc

_NC = 2
_NS = 16
_NW = _NC * _NS
_L = 16

_EMBED = 64
_BLK = 64
_C = 32768


def _tc_scores_body(x_ref, w_ref, o_ref):
    x = x_ref[...]
    w = w_ref[...]
    s = jnp.sum(x * w, axis=0)
    o_ref[...] = s.reshape(_C // 128, 128)


def _tc_scores(tab_t, w):
    n = tab_t.shape[1]
    grid = (n + _C - 1) // _C
    return pl.pallas_call(
        _tc_scores_body,
        grid=(grid,),
        in_specs=[
            pl.BlockSpec((_EMBED, _C), lambda i: (0, i)),
            pl.BlockSpec((_EMBED, 1), lambda i: (0, 0)),
        ],
        out_specs=pl.BlockSpec((_C // 128, 128), lambda i: (i, 0)),
        out_shape=jax.ShapeDtypeStruct((grid * (_C // 128), 128), jnp.float32),
    )(tab_t, w)


def _sc_body(qu_r, tu_r, qm_r, tm_r, su, sm, brep, out,
             qu_v, tu_v, qm_v, tm_v, rows_u, rows_m, out_v, b_v,
             sem_u0, sem_u1, sem_m0, sem_m1,
             *, b_per_w, n_blocks):
    wid = lax.axis_index("s") * _NC + lax.axis_index("c")
    chunks_per_blk = _BLK // _L

    pltpu.sync_copy(qu_r.at[wid], qu_v)
    pltpu.sync_copy(tu_r.at[wid], tu_v)
    pltpu.sync_copy(qm_r.at[wid], qm_v)
    pltpu.sync_copy(tm_r.at[wid], tm_v)
    pltpu.sync_copy(brep, b_v)

    sems_u = (sem_u0, sem_u1)
    sems_m = (sem_m0, sem_m1)

    def fire(j):
        s = j % 2
        blk = pl.ds(j * _BLK, _BLK)
        pltpu.async_copy(su.at[qu_v.at[blk]], rows_u.at[s], sems_u[s])
        pltpu.async_copy(sm.at[qm_v.at[blk]], rows_m.at[s], sems_m[s])

    def drain(j):
        s = j % 2
        pltpu.make_async_copy(su.at[pl.ds(0, _BLK)], rows_u.at[s],
                              sems_u[s]).wait()
        pltpu.make_async_copy(sm.at[pl.ds(0, _BLK)], rows_m.at[s],
                              sems_m[s]).wait()

    row16 = lax.iota(jnp.int32, _L)
    bias = b_v[...]

    fire(0)
    fire(1)
    for j in range(n_blocks):
        s = j % 2
        drain(j)
        ru = rows_u.at[s]
        rm = rows_m.at[s]

        def chunk(k, carry, *, j=j, ru=ru, rm=rm):
            g = j * chunks_per_blk + k
            rid = k * _L + row16
            t_u = tu_v[pl.ds(g * _L, _L)]
            t_m = tm_v[pl.ds(g * _L, _L)]
            vu = plsc.load_gather(ru, [rid, t_u])
            vm = plsc.load_gather(rm, [rid, t_m])
            out_v[pl.ds(g * _L, _L)] = vu + vm + bias
            return carry

        lax.fori_loop(0, chunks_per_blk, chunk, 0)
        if j + 2 < n_blocks:
            fire(j + 2)

    pltpu.sync_copy(out_v, out.at[wid])


def kernel(users, movies, user_table, movie_table, W, b):
    B = users.shape[0]
    assert B % (_NW * _BLK) == 0
    b_per_w = B // _NW
    n_blocks = b_per_w // _BLK

    users = users.astype(jnp.int32)
    movies = movies.astype(jnp.int32)
    qu = (users >> 7).reshape(_NW, b_per_w)
    tu = (users & 127).reshape(_NW, b_per_w)
    qm = (movies >> 7).reshape(_NW, b_per_w)
    tm = (movies & 127).reshape(_NW, b_per_w)

    utab_t = user_table.T
    mtab_t = movie_table.T
    wu = W.reshape(-1)[:_EMBED].astype(jnp.float32).reshape(_EMBED, 1)
    wm = W.reshape(-1)[_EMBED:].astype(jnp.float32).reshape(_EMBED, 1)

    scores_u = _tc_scores(utab_t, wu)
    scores_m = _tc_scores(mtab_t, wm)

    brep = jnp.full((_L,), b.reshape(()), dtype=jnp.float32)

    mesh = plsc.VectorSubcoreMesh(core_axis_name="c", subcore_axis_name="s")
    body = functools.partial(_sc_body, b_per_w=b_per_w, n_blocks=n_blocks)
    run = pl.kernel(
        body,
        out_type=jax.ShapeDtypeStruct((_NW, b_per_w), jnp.float32),
        mesh=mesh,
        compiler_params=pltpu.CompilerParams(needs_layout_passes=False),
        scratch_types=[
            pltpu.VMEM((b_per_w,), jnp.int32),
            pltpu.VMEM((b_per_w,), jnp.int32),
            pltpu.VMEM((b_per_w,), jnp.int32),
            pltpu.VMEM((b_per_w,), jnp.int32),
            pltpu.VMEM((2, _BLK, 128), jnp.float32),
            pltpu.VMEM((2, _BLK, 128), jnp.float32),
            pltpu.VMEM((b_per_w,), jnp.float32),
            pltpu.VMEM((_L,), jnp.float32),
            pltpu.SemaphoreType.DMA,
            pltpu.SemaphoreType.DMA,
            pltpu.SemaphoreType.DMA,
            pltpu.SemaphoreType.DMA,
        ],
    )
    out = run(qu, tu, qm, tm, scores_u, scores_m, brep)
    return out.reshape(B, 1)
